# baseline (speedup 1.0000x reference)
import functools

import jax
import jax.numpy as jnp
from jax import lax
from jax.experimental import pallas as pl
from jax.experimental.pallas import tpu as pltpu
from jax.experimental.pallas import tpu_sc as plsc

V = 100000
D = 64
CTX = 20
B = 1024

_info = plsc.get_sparse_core_info()
NC, NS, L = _info.num_cores, _info.num_subcores, _info.num_lanes
NW = NC * NS
PER_W = B // NW
ROWS_W = PER_W * CTX
CHUNK = 128
NCHUNK = ROWS_W // CHUNK

_mesh = plsc.VectorSubcoreMesh(core_axis_name="c", subcore_axis_name="s")


@functools.partial(
    pl.kernel,
    out_type=jax.ShapeDtypeStruct((B, D), jnp.float32),
    mesh=_mesh,
    scratch_types=[
        pltpu.VMEM((NCHUNK, CHUNK), jnp.int32),
        pltpu.VMEM((ROWS_W, D), jnp.float32),
        pltpu.VMEM((PER_W, D), jnp.float32),
        pltpu.SemaphoreType.DMA,
    ],
    compiler_params=pltpu.CompilerParams(use_tc_tiling_on_sc=False),
)
def _sc_gather_mean(xt_hbm, table_hbm, ctx_hbm, idx_v, rows_v, acc_v, sem):
    wid = lax.axis_index("s") * NC + lax.axis_index("c")

    pltpu.sync_copy(xt_hbm.at[wid], idx_v)
    copies = []
    for j in range(NCHUNK):
        copies.append(
            pltpu.async_copy(
                table_hbm.at[idx_v.at[j]],
                rows_v.at[pl.ds(j * CHUNK, CHUNK)],
                sem,
            )
        )
    for cp in copies:
        cp.wait()

    scale = jnp.full((L,), 1.0 / CTX, dtype=jnp.float32)

    def body(i, carry):
        base = i * CTX
        for c in range(D // L):
            v = rows_v[base, pl.ds(c * L, L)]
            for j in range(1, CTX):
                v = v + rows_v[base + j, pl.ds(c * L, L)]
            acc_v[i, pl.ds(c * L, L)] = v * scale
        return carry

    lax.fori_loop(0, PER_W, body, 0)

    pltpu.sync_copy(acc_v, ctx_hbm.at[pl.ds(wid * PER_W, PER_W)])


NV_BLK = 2048
GRID_V = (V + NV_BLK - 1) // NV_BLK


def _proj_body(ctx_ref, w_ref, b_ref, out_ref):
    out_ref[...] = (
        lax.dot_general(
            ctx_ref[...],
            w_ref[...],
            (((1,), (1,)), ((), ())),
            preferred_element_type=jnp.float32,
        )
        + b_ref[...]
    )


_proj = pl.pallas_call(
    _proj_body,
    grid=(GRID_V,),
    in_specs=[
        pl.BlockSpec((B, D), lambda i: (0, 0)),
        pl.BlockSpec((NV_BLK, D), lambda i: (i, 0)),
        pl.BlockSpec((1, NV_BLK), lambda i: (0, i)),
    ],
    out_specs=pl.BlockSpec((B, NV_BLK), lambda i: (0, i)),
    out_shape=jax.ShapeDtypeStruct((B, V), jnp.float32),
)


@jax.jit
def kernel(x, emb_table, W, b):
    xt = x.T.reshape(NW, NCHUNK, CHUNK)
    ctx = _sc_gather_mean(xt, emb_table)
    return _proj(ctx, W, b.reshape(1, V))


# speedup vs baseline: 2.7427x; 2.7427x over previous
import functools

import jax
import jax.numpy as jnp
from jax import lax
from jax.experimental import pallas as pl
from jax.experimental.pallas import tpu as pltpu
from jax.experimental.pallas import tpu_sc as plsc

V = 100000
D = 64
CTX = 20
B = 1024

_info = plsc.get_sparse_core_info()
NC, NS, L = _info.num_cores, _info.num_subcores, _info.num_lanes
NW = NC * NS
PER_W = B // NW
ROWS_W = PER_W * CTX
CHUNK = 128
NCHUNK = ROWS_W // CHUNK

_mesh = plsc.VectorSubcoreMesh(core_axis_name="c", subcore_axis_name="s")


@functools.partial(
    pl.kernel,
    out_type=jax.ShapeDtypeStruct((B, D), jnp.float32),
    mesh=_mesh,
    scratch_types=[
        pltpu.VMEM((NCHUNK, CHUNK), jnp.int32),
        pltpu.VMEM((ROWS_W, D), jnp.float32),
        pltpu.VMEM((PER_W, D), jnp.float32),
        pltpu.SemaphoreType.DMA,
    ],
    compiler_params=pltpu.CompilerParams(use_tc_tiling_on_sc=False),
)
def _sc_gather_mean(xt_hbm, table_hbm, ctx_hbm, idx_v, rows_v, acc_v, sem):
    wid = lax.axis_index("s") * NC + lax.axis_index("c")

    pltpu.sync_copy(xt_hbm.at[wid], idx_v)
    copies = []
    for j in range(NCHUNK):
        copies.append(
            pltpu.async_copy(
                table_hbm.at[idx_v.at[j]],
                rows_v.at[pl.ds(j * CHUNK, CHUNK)],
                sem,
            )
        )
    for cp in copies:
        cp.wait()

    scale = jnp.full((L,), 1.0 / CTX, dtype=jnp.float32)

    def body(i, carry):
        base = i * CTX
        for c in range(D // L):
            v = rows_v[base, pl.ds(c * L, L)]
            for j in range(1, CTX):
                v = v + rows_v[base + j, pl.ds(c * L, L)]
            acc_v[i, pl.ds(c * L, L)] = v * scale
        return carry

    lax.fori_loop(0, PER_W, body, 0)

    pltpu.sync_copy(acc_v, ctx_hbm.at[pl.ds(wid * PER_W, PER_W)])


NV_BLK = 2048
GRID_V = (V + NV_BLK - 1) // NV_BLK


def _proj_body(ctx_ref, wt_ref, b_ref, out_ref):
    mm = lax.dot_general(
        wt_ref[...],
        ctx_ref[...],
        (((0,), (1,)), ((), ())),
        preferred_element_type=jnp.float32,
    )
    ones = jnp.ones((1, B), dtype=jnp.float32)
    bias = lax.dot_general(
        b_ref[...],
        ones,
        (((0,), (0,)), ((), ())),
        preferred_element_type=jnp.float32,
    )
    out_ref[...] = mm + bias


_proj = pl.pallas_call(
    _proj_body,
    grid=(GRID_V,),
    in_specs=[
        pl.BlockSpec((B, D), lambda i: (0, 0)),
        pl.BlockSpec((D, NV_BLK), lambda i: (0, i)),
        pl.BlockSpec((1, NV_BLK), lambda i: (0, i)),
    ],
    out_specs=pl.BlockSpec((NV_BLK, B), lambda i: (i, 0)),
    out_shape=jax.ShapeDtypeStruct((V, B), jnp.float32),
)


@jax.jit
def kernel(x, emb_table, W, b):
    xt = x.T.reshape(NW, NCHUNK, CHUNK)
    ctx = _sc_gather_mean(xt, emb_table)
    out_t = _proj(ctx, W.T, b.reshape(1, V))
    return out_t.T


# speedup vs baseline: 2.8253x; 1.0301x over previous
import functools

import jax
import jax.numpy as jnp
from jax import lax
from jax.experimental import pallas as pl
from jax.experimental.pallas import tpu as pltpu
from jax.experimental.pallas import tpu_sc as plsc

V = 100000
D = 64
CTX = 20
B = 1024

_info = plsc.get_sparse_core_info()
NC, NS, L = _info.num_cores, _info.num_subcores, _info.num_lanes
NW = NC * NS
PER_W = B // NW
ROWS_W = PER_W * CTX
D_PAD = 128
CHUNK = 128
NCHUNK = ROWS_W // CHUNK

_mesh = plsc.VectorSubcoreMesh(core_axis_name="c", subcore_axis_name="s")


@functools.partial(
    pl.kernel,
    out_type=jax.ShapeDtypeStruct((B, D), jnp.float32),
    mesh=_mesh,
    scratch_types=[
        pltpu.VMEM((NCHUNK, CHUNK), jnp.int32),
        pltpu.VMEM((ROWS_W, D_PAD), jnp.float32),
        pltpu.VMEM((PER_W, D), jnp.float32),
        pltpu.SemaphoreType.DMA,
    ],
    compiler_params=pltpu.CompilerParams(use_tc_tiling_on_sc=False),
)
def _sc_gather_mean(xt_hbm, table_hbm, ctx_hbm, idx_v, rows_v, acc_v, sem):
    wid = lax.axis_index("s") * NC + lax.axis_index("c")

    pltpu.sync_copy(xt_hbm.at[wid], idx_v)
    copies = []
    for j in range(NCHUNK):
        copies.append(
            pltpu.async_copy(
                table_hbm.at[idx_v.at[j]],
                rows_v.at[pl.ds(j * CHUNK, CHUNK)],
                sem,
            )
        )
    for cp in copies:
        cp.wait()

    scale = jnp.full((L,), 1.0 / CTX, dtype=jnp.float32)

    def body(i, carry):
        base = i * CTX
        for c in range(D // L):
            v = rows_v[base, pl.ds(c * L, L)]
            for j in range(1, CTX):
                v = v + rows_v[base + j, pl.ds(c * L, L)]
            acc_v[i, pl.ds(c * L, L)] = v * scale
        return carry

    lax.fori_loop(0, PER_W, body, 0)

    pltpu.sync_copy(acc_v, ctx_hbm.at[pl.ds(wid * PER_W, PER_W)])


NV_BLK = 2048
GRID_V = (V + NV_BLK - 1) // NV_BLK


def _proj_body(ctx_ref, wt_ref, b_ref, out_ref):
    mm = lax.dot_general(
        wt_ref[...],
        ctx_ref[...],
        (((0,), (1,)), ((), ())),
        preferred_element_type=jnp.float32,
    )
    ones = jnp.ones((1, B), dtype=jnp.float32)
    bias = lax.dot_general(
        b_ref[...],
        ones,
        (((0,), (0,)), ((), ())),
        preferred_element_type=jnp.float32,
    )
    out_ref[...] = mm + bias


_proj = pl.pallas_call(
    _proj_body,
    grid=(GRID_V,),
    in_specs=[
        pl.BlockSpec((B, D), lambda i: (0, 0)),
        pl.BlockSpec((D, NV_BLK), lambda i: (0, i)),
        pl.BlockSpec((1, NV_BLK), lambda i: (0, i)),
    ],
    out_specs=pl.BlockSpec((NV_BLK, B), lambda i: (i, 0)),
    out_shape=jax.ShapeDtypeStruct((V, B), jnp.float32),
)


@jax.jit
def kernel(x, emb_table, W, b):
    xt = x.T.reshape(NW, NCHUNK, CHUNK)
    emb_pad = jnp.pad(emb_table, ((0, 0), (0, D_PAD - D)))
    ctx = _sc_gather_mean(xt, emb_pad)
    out_t = _proj(ctx, W.T, b.reshape(1, V))
    return out_t.T


# speedup vs baseline: 2.9007x; 1.0267x over previous
import functools

import jax
import jax.numpy as jnp
from jax import lax
from jax.experimental import pallas as pl
from jax.experimental.pallas import tpu as pltpu
from jax.experimental.pallas import tpu_sc as plsc

V = 100000
D = 64
CTX = 20
B = 1024

_info = plsc.get_sparse_core_info()
NC, NS, L = _info.num_cores, _info.num_subcores, _info.num_lanes
NW = NC * NS
PER_W = B // NW
ROWS_W = PER_W * CTX
D_PAD = 128
CHUNK = 128
NCHUNK = ROWS_W // CHUNK

_mesh = plsc.VectorSubcoreMesh(core_axis_name="c", subcore_axis_name="s")


@functools.partial(
    pl.kernel,
    out_type=jax.ShapeDtypeStruct((B, D), jnp.float32),
    mesh=_mesh,
    scratch_types=[
        pltpu.VMEM((NCHUNK, CHUNK), jnp.int32),
        pltpu.VMEM((ROWS_W, D_PAD), jnp.float32),
        pltpu.VMEM((PER_W, D), jnp.float32),
        pltpu.SemaphoreType.DMA,
    ],
    compiler_params=pltpu.CompilerParams(use_tc_tiling_on_sc=False),
)
def _sc_gather_mean(xt_hbm, table_hbm, ctx_hbm, idx_v, rows_v, acc_v, sem):
    wid = lax.axis_index("s") * NC + lax.axis_index("c")

    pltpu.sync_copy(xt_hbm.at[wid], idx_v)
    copies = []
    for j in range(NCHUNK):
        copies.append(
            pltpu.async_copy(
                table_hbm.at[idx_v.at[j]],
                rows_v.at[pl.ds(j * CHUNK, CHUNK)],
                sem,
            )
        )
    for cp in copies:
        cp.wait()

    scale = jnp.full((L,), 1.0 / CTX, dtype=jnp.float32)

    def body(i, carry):
        base = i * CTX
        for c in range(D // L):
            v = rows_v[base, pl.ds(c * L, L)]
            for j in range(1, CTX):
                v = v + rows_v[base + j, pl.ds(c * L, L)]
            acc_v[i, pl.ds(c * L, L)] = v * scale
        return carry

    lax.fori_loop(0, PER_W, body, 0)

    pltpu.sync_copy(acc_v, ctx_hbm.at[pl.ds(wid * PER_W, PER_W)])


NT_BLK = 2048
GRID_T = (V + NT_BLK - 1) // NT_BLK


def _tr_body(et_ref, out_ref):
    out_ref[:, :D] = et_ref[...].T
    out_ref[:, D:] = jnp.zeros((NT_BLK, D_PAD - D), jnp.float32)


_transpose_pad = pl.pallas_call(
    _tr_body,
    grid=(GRID_T,),
    in_specs=[pl.BlockSpec((D, NT_BLK), lambda i: (0, i))],
    out_specs=pl.BlockSpec((NT_BLK, D_PAD), lambda i: (i, 0)),
    out_shape=jax.ShapeDtypeStruct((V, D_PAD), jnp.float32),
)


NV_BLK = 2048
GRID_V = (V + NV_BLK - 1) // NV_BLK


def _proj_body(ctx_ref, wt_ref, b_ref, out_ref):
    mm = lax.dot_general(
        wt_ref[...],
        ctx_ref[...],
        (((0,), (1,)), ((), ())),
        preferred_element_type=jnp.float32,
    )
    ones = jnp.ones((1, B), dtype=jnp.float32)
    bias = lax.dot_general(
        b_ref[...],
        ones,
        (((0,), (0,)), ((), ())),
        preferred_element_type=jnp.float32,
    )
    out_ref[...] = mm + bias


_proj = pl.pallas_call(
    _proj_body,
    grid=(GRID_V,),
    in_specs=[
        pl.BlockSpec((B, D), lambda i: (0, 0)),
        pl.BlockSpec((D, NV_BLK), lambda i: (0, i)),
        pl.BlockSpec((1, NV_BLK), lambda i: (0, i)),
    ],
    out_specs=pl.BlockSpec((NV_BLK, B), lambda i: (i, 0)),
    out_shape=jax.ShapeDtypeStruct((V, B), jnp.float32),
)


@jax.jit
def kernel(x, emb_table, W, b):
    xt = x.T.reshape(NW, NCHUNK, CHUNK)
    emb_pad = _transpose_pad(emb_table.T)
    ctx = _sc_gather_mean(xt, emb_pad)
    out_t = _proj(ctx, W.T, b.reshape(1, V))
    return out_t.T


# speedup vs baseline: 3.1951x; 1.1015x over previous
import functools

import jax
import jax.numpy as jnp
from jax import lax
from jax.experimental import pallas as pl
from jax.experimental.pallas import tpu as pltpu
from jax.experimental.pallas import tpu_sc as plsc

V = 100000
D = 64
CTX = 20
B = 1024

_info = plsc.get_sparse_core_info()
NC, NS, L = _info.num_cores, _info.num_subcores, _info.num_lanes
NW = NC * NS
PER_W = B // NW
ROWS_W = PER_W * CTX
D_PAD = 128
CHUNK = 128
NCHUNK = ROWS_W // CHUNK

_mesh = plsc.VectorSubcoreMesh(core_axis_name="c", subcore_axis_name="s")


@functools.partial(
    pl.kernel,
    out_type=jax.ShapeDtypeStruct((B, D), jnp.float32),
    mesh=_mesh,
    scratch_types=[
        pltpu.VMEM((NCHUNK, CHUNK), jnp.int32),
        pltpu.VMEM((ROWS_W, D_PAD), jnp.float32),
        pltpu.VMEM((PER_W, D), jnp.float32),
        pltpu.SemaphoreType.DMA,
    ],
    compiler_params=pltpu.CompilerParams(use_tc_tiling_on_sc=False),
)
def _sc_gather_mean(xt_hbm, table_hbm, ctx_hbm, idx_v, rows_v, acc_v, sem):
    wid = lax.axis_index("s") * NC + lax.axis_index("c")

    pltpu.sync_copy(xt_hbm.at[wid], idx_v)
    copies = []
    for j in range(NCHUNK):
        copies.append(
            pltpu.async_copy(
                table_hbm.at[idx_v.at[j]],
                rows_v.at[pl.ds(j * CHUNK, CHUNK)],
                sem,
            )
        )
    for cp in copies:
        cp.wait()

    scale = jnp.full((L,), 1.0 / CTX, dtype=jnp.float32)

    def body(i, carry):
        base = i * CTX
        for c in range(D // L):
            v = rows_v[base, pl.ds(c * L, L)]
            for j in range(1, CTX):
                v = v + rows_v[base + j, pl.ds(c * L, L)]
            acc_v[i, pl.ds(c * L, L)] = v * scale
        return carry

    lax.fori_loop(0, PER_W, body, 0)

    pltpu.sync_copy(acc_v, ctx_hbm.at[pl.ds(wid * PER_W, PER_W)])


NT_BLK = 8192
GRID_T = (V + NT_BLK - 1) // NT_BLK


def _tr_body(et_ref, out_ref):
    out_ref[:, :D] = et_ref[...].T
    out_ref[:, D:] = jnp.zeros((NT_BLK, D_PAD - D), jnp.float32)


_transpose_pad = pl.pallas_call(
    _tr_body,
    grid=(GRID_T,),
    in_specs=[pl.BlockSpec((D, NT_BLK), lambda i: (0, i))],
    out_specs=pl.BlockSpec((NT_BLK, D_PAD), lambda i: (i, 0)),
    out_shape=jax.ShapeDtypeStruct((V, D_PAD), jnp.float32),
)


NV_BLK = 2048
GRID_V = (V + NV_BLK - 1) // NV_BLK


def _proj_body(ctx_ref, wt_ref, b_ref, out_ref):
    mm = lax.dot_general(
        wt_ref[...],
        ctx_ref[...],
        (((0,), (1,)), ((), ())),
        preferred_element_type=jnp.float32,
    )
    ones = jnp.ones((1, B), dtype=jnp.float32)
    bias = lax.dot_general(
        b_ref[...],
        ones,
        (((0,), (0,)), ((), ())),
        preferred_element_type=jnp.float32,
    )
    out_ref[...] = mm + bias


_proj = pl.pallas_call(
    _proj_body,
    grid=(GRID_V,),
    in_specs=[
        pl.BlockSpec((B, D), lambda i: (0, 0)),
        pl.BlockSpec((D, NV_BLK), lambda i: (0, i)),
        pl.BlockSpec((1, NV_BLK), lambda i: (0, i)),
    ],
    out_specs=pl.BlockSpec((NV_BLK, B), lambda i: (i, 0)),
    out_shape=jax.ShapeDtypeStruct((V, B), jnp.float32),
)


@jax.jit
def kernel(x, emb_table, W, b):
    xt = x.T.reshape(NW, NCHUNK, CHUNK)
    emb_pad = _transpose_pad(emb_table.T)
    ctx = _sc_gather_mean(xt, emb_pad)
    out_t = _proj(ctx, W.T, b.reshape(1, V))
    return out_t.T


# speedup vs baseline: 3.2232x; 1.0088x over previous
import functools

import jax
import jax.numpy as jnp
from jax import lax
from jax.experimental import pallas as pl
from jax.experimental.pallas import tpu as pltpu
from jax.experimental.pallas import tpu_sc as plsc

V = 100000
D = 64
CTX = 20
B = 1024

_info = plsc.get_sparse_core_info()
NC, NS, L = _info.num_cores, _info.num_subcores, _info.num_lanes
NW = NC * NS
PER_W = B // NW
ROWS_W = PER_W * CTX
D_PAD = 128
CHUNK = 128
NCHUNK = ROWS_W // CHUNK

_mesh = plsc.VectorSubcoreMesh(core_axis_name="c", subcore_axis_name="s")


@functools.partial(
    pl.kernel,
    out_type=jax.ShapeDtypeStruct((B, D), jnp.float32),
    mesh=_mesh,
    scratch_types=[
        pltpu.VMEM((NCHUNK, CHUNK), jnp.int32),
        pltpu.VMEM((ROWS_W, D_PAD), jnp.float32),
        pltpu.VMEM((PER_W, D), jnp.float32),
        pltpu.SemaphoreType.DMA,
    ],
    compiler_params=pltpu.CompilerParams(use_tc_tiling_on_sc=False),
)
def _sc_gather_mean(xt_hbm, table_hbm, ctx_hbm, idx_v, rows_v, acc_v, sem):
    wid = lax.axis_index("s") * NC + lax.axis_index("c")

    pltpu.sync_copy(xt_hbm.at[wid], idx_v)
    copies = []
    for j in range(NCHUNK):
        copies.append(
            pltpu.async_copy(
                table_hbm.at[idx_v.at[j]],
                rows_v.at[pl.ds(j * CHUNK, CHUNK)],
                sem,
            )
        )
    for cp in copies:
        cp.wait()

    scale = jnp.full((L,), 1.0 / CTX, dtype=jnp.float32)

    def body(i, carry):
        base = i * CTX
        for c in range(D // L):
            v = rows_v[base, pl.ds(c * L, L)]
            for j in range(1, CTX):
                v = v + rows_v[base + j, pl.ds(c * L, L)]
            acc_v[i, pl.ds(c * L, L)] = v * scale
        return carry

    lax.fori_loop(0, PER_W, body, 0)

    pltpu.sync_copy(acc_v, ctx_hbm.at[pl.ds(wid * PER_W, PER_W)])


NT_BLK = 16384
GRID_T = (V + NT_BLK - 1) // NT_BLK


def _tr_body(et_ref, out_ref):
    out_ref[:, :D] = et_ref[...].T


_transpose_pad = pl.pallas_call(
    _tr_body,
    grid=(GRID_T,),
    in_specs=[pl.BlockSpec((D, NT_BLK), lambda i: (0, i))],
    out_specs=pl.BlockSpec((NT_BLK, D_PAD), lambda i: (i, 0)),
    out_shape=jax.ShapeDtypeStruct((V, D_PAD), jnp.float32),
)


NV_BLK = 2048
GRID_V = (V + NV_BLK - 1) // NV_BLK


def _proj_body(ctx_ref, wt_ref, b_ref, out_ref):
    mm = lax.dot_general(
        wt_ref[...],
        ctx_ref[...],
        (((0,), (1,)), ((), ())),
        preferred_element_type=jnp.float32,
    )
    ones = jnp.ones((1, B), dtype=jnp.float32)
    bias = lax.dot_general(
        b_ref[...],
        ones,
        (((0,), (0,)), ((), ())),
        preferred_element_type=jnp.float32,
    )
    out_ref[...] = mm + bias


_proj = pl.pallas_call(
    _proj_body,
    grid=(GRID_V,),
    in_specs=[
        pl.BlockSpec((B, D), lambda i: (0, 0)),
        pl.BlockSpec((D, NV_BLK), lambda i: (0, i)),
        pl.BlockSpec((1, NV_BLK), lambda i: (0, i)),
    ],
    out_specs=pl.BlockSpec((NV_BLK, B), lambda i: (i, 0)),
    out_shape=jax.ShapeDtypeStruct((V, B), jnp.float32),
)


@jax.jit
def kernel(x, emb_table, W, b):
    xt = x.T.reshape(NW, NCHUNK, CHUNK)
    emb_pad = _transpose_pad(emb_table.T)
    ctx = _sc_gather_mean(xt, emb_pad)
    out_t = _proj(ctx, W.T, b.reshape(1, V))
    return out_t.T


# speedup vs baseline: 3.2654x; 1.0131x over previous
import functools

import jax
import jax.numpy as jnp
from jax import lax
from jax.experimental import pallas as pl
from jax.experimental.pallas import tpu as pltpu
from jax.experimental.pallas import tpu_sc as plsc

V = 100000
D = 64
CTX = 20
B = 1024

_info = plsc.get_sparse_core_info()
NC, NS, L = _info.num_cores, _info.num_subcores, _info.num_lanes
NW = NC * NS
PER_W = B // NW
ROWS_W = PER_W * CTX
D_PAD = 128
CHUNK = 128
NCHUNK = ROWS_W // CHUNK

_mesh = plsc.VectorSubcoreMesh(core_axis_name="c", subcore_axis_name="s")


@functools.partial(
    pl.kernel,
    out_type=jax.ShapeDtypeStruct((B, D_PAD), jnp.float32),
    mesh=_mesh,
    scratch_types=[
        pltpu.VMEM((NCHUNK, CHUNK), jnp.int32),
        pltpu.VMEM((ROWS_W, D_PAD), jnp.float32),
        pltpu.VMEM((PER_W, D_PAD), jnp.float32),
        pltpu.SemaphoreType.DMA,
    ],
    compiler_params=pltpu.CompilerParams(use_tc_tiling_on_sc=False),
)
def _sc_gather_mean(xt_hbm, table_hbm, ctx_hbm, idx_v, rows_v, acc_v, sem):
    wid = lax.axis_index("s") * NC + lax.axis_index("c")

    pltpu.sync_copy(xt_hbm.at[wid], idx_v)
    copies = []
    for j in range(NCHUNK):
        copies.append(
            pltpu.async_copy(
                table_hbm.at[idx_v.at[j]],
                rows_v.at[pl.ds(j * CHUNK, CHUNK)],
                sem,
            )
        )
    for cp in copies:
        cp.wait()

    scale = jnp.full((L,), 1.0 / CTX, dtype=jnp.float32)

    def body(i, carry):
        base = i * CTX
        for c in range(D // L):
            v = rows_v[base, pl.ds(c * L, L)]
            for j in range(1, CTX):
                v = v + rows_v[base + j, pl.ds(c * L, L)]
            acc_v[i, pl.ds(c * L, L)] = v * scale
        return carry

    lax.fori_loop(0, PER_W, body, 0)

    pltpu.sync_copy(acc_v, ctx_hbm.at[pl.ds(wid * PER_W, PER_W)])


NT_BLK = 16384
GRID_T = (V + NT_BLK - 1) // NT_BLK


def _tr_body(et_ref, out_ref):
    out_ref[:, :D] = et_ref[...].T


_transpose_pad = pl.pallas_call(
    _tr_body,
    grid=(GRID_T,),
    in_specs=[pl.BlockSpec((D, NT_BLK), lambda i: (0, i))],
    out_specs=pl.BlockSpec((NT_BLK, D_PAD), lambda i: (i, 0)),
    out_shape=jax.ShapeDtypeStruct((V, D_PAD), jnp.float32),
)


NV_BLK = 2048
GRID_V = (V + NV_BLK - 1) // NV_BLK


def _proj_body(ctx_ref, wt_ref, b_ref, out_ref):
    mm = lax.dot_general(
        wt_ref[...],
        ctx_ref[:, :D],
        (((0,), (1,)), ((), ())),
        preferred_element_type=jnp.float32,
    )
    ones = jnp.ones((1, B), dtype=jnp.float32)
    bias = lax.dot_general(
        b_ref[...],
        ones,
        (((0,), (0,)), ((), ())),
        preferred_element_type=jnp.float32,
    )
    out_ref[...] = mm + bias


_proj = pl.pallas_call(
    _proj_body,
    grid=(GRID_V,),
    in_specs=[
        pl.BlockSpec((B, D_PAD), lambda i: (0, 0)),
        pl.BlockSpec((D, NV_BLK), lambda i: (0, i)),
        pl.BlockSpec((1, NV_BLK), lambda i: (0, i)),
    ],
    out_specs=pl.BlockSpec((NV_BLK, B), lambda i: (i, 0)),
    out_shape=jax.ShapeDtypeStruct((V, B), jnp.float32),
)


@jax.jit
def kernel(x, emb_table, W, b):
    xt = x.T.reshape(NW, NCHUNK, CHUNK)
    emb_pad = _transpose_pad(emb_table.T)
    ctx = _sc_gather_mean(xt, emb_pad)
    out_t = _proj(ctx, W.T, b.reshape(1, V))
    return out_t.T


# speedup vs baseline: 3.2791x; 1.0042x over previous
import functools

import jax
import jax.numpy as jnp
from jax import lax
from jax.experimental import pallas as pl
from jax.experimental.pallas import tpu as pltpu
from jax.experimental.pallas import tpu_sc as plsc

V = 100000
D = 64
CTX = 20
B = 1024

_info = plsc.get_sparse_core_info()
NC, NS, L = _info.num_cores, _info.num_subcores, _info.num_lanes
NW = NC * NS
PER_W = B // NW
ROWS_W = PER_W * CTX
D_PAD = 128
CHUNK = 128
NCHUNK = ROWS_W // CHUNK

_mesh = plsc.VectorSubcoreMesh(core_axis_name="c", subcore_axis_name="s")


@functools.partial(
    pl.kernel,
    out_type=jax.ShapeDtypeStruct((B, D_PAD), jnp.float32),
    mesh=_mesh,
    scratch_types=[
        pltpu.VMEM((NCHUNK, CHUNK), jnp.int32),
        pltpu.VMEM((ROWS_W, D_PAD), jnp.float32),
        pltpu.VMEM((PER_W, D_PAD), jnp.float32),
        pltpu.SemaphoreType.DMA,
    ],
    compiler_params=pltpu.CompilerParams(use_tc_tiling_on_sc=False),
)
def _sc_gather_mean(xt_hbm, table_hbm, ctx_hbm, idx_v, rows_v, acc_v, sem):
    wid = lax.axis_index("s") * NC + lax.axis_index("c")

    pltpu.sync_copy(xt_hbm.at[wid], idx_v)
    copies = []
    for j in range(NCHUNK):
        copies.append(
            pltpu.async_copy(
                table_hbm.at[idx_v.at[j]],
                rows_v.at[pl.ds(j * CHUNK, CHUNK)],
                sem,
            )
        )
    scale = jnp.full((L,), 1.0 / CTX, dtype=jnp.float32)

    def body(i, carry):
        base = i * CTX
        for c in range(D // L):
            v = rows_v[base, pl.ds(c * L, L)]
            for j in range(1, CTX):
                v = v + rows_v[base + j, pl.ds(c * L, L)]
            acc_v[i, pl.ds(c * L, L)] = v * scale
        return carry

    done = 0
    for j in range(NCHUNK):
        copies[j].wait()
        upto = (CHUNK * (j + 1)) // CTX
        lax.fori_loop(done, upto, body, 0)
        done = upto

    pltpu.sync_copy(acc_v, ctx_hbm.at[pl.ds(wid * PER_W, PER_W)])


NT_BLK = 16384
GRID_T = (V + NT_BLK - 1) // NT_BLK


def _tr_body(et_ref, out_ref):
    out_ref[:, :D] = et_ref[...].T


_transpose_pad = pl.pallas_call(
    _tr_body,
    grid=(GRID_T,),
    in_specs=[pl.BlockSpec((D, NT_BLK), lambda i: (0, i))],
    out_specs=pl.BlockSpec((NT_BLK, D_PAD), lambda i: (i, 0)),
    out_shape=jax.ShapeDtypeStruct((V, D_PAD), jnp.float32),
)


NV_BLK = 2048
GRID_V = (V + NV_BLK - 1) // NV_BLK


def _proj_body(ctx_ref, wt_ref, b_ref, out_ref):
    mm = lax.dot_general(
        wt_ref[...],
        ctx_ref[:, :D],
        (((0,), (1,)), ((), ())),
        preferred_element_type=jnp.float32,
    )
    ones = jnp.ones((1, B), dtype=jnp.float32)
    bias = lax.dot_general(
        b_ref[...],
        ones,
        (((0,), (0,)), ((), ())),
        preferred_element_type=jnp.float32,
    )
    out_ref[...] = mm + bias


_proj = pl.pallas_call(
    _proj_body,
    grid=(GRID_V,),
    in_specs=[
        pl.BlockSpec((B, D_PAD), lambda i: (0, 0)),
        pl.BlockSpec((D, NV_BLK), lambda i: (0, i)),
        pl.BlockSpec((1, NV_BLK), lambda i: (0, i)),
    ],
    out_specs=pl.BlockSpec((NV_BLK, B), lambda i: (i, 0)),
    out_shape=jax.ShapeDtypeStruct((V, B), jnp.float32),
)


@jax.jit
def kernel(x, emb_table, W, b):
    xt = x.T.reshape(NW, NCHUNK, CHUNK)
    emb_pad = _transpose_pad(emb_table.T)
    ctx = _sc_gather_mean(xt, emb_pad)
    out_t = _proj(ctx, W.T, b.reshape(1, V))
    return out_t.T
